# baseline (speedup 1.0000x reference)
import jax
import jax.numpy as jnp
from jax import lax
from jax.experimental import pallas as pl
from jax.experimental.pallas import tpu as pltpu
from jax.experimental.pallas import tpu_sc as plsc

NUM_VERTICES = 1_000_000
NUM_TRIANGLES = 2_000_000
CH = 10000
NCH = NUM_TRIANGLES // CH
NW = 32


def _gather_body(t0, t1, t2, v0, v1, v2, out_hbm, idx_v, row_v, sem_i, sem_g):
    wid = lax.axis_index("s") * 2 + lax.axis_index("c")
    tri_planes = (t0, t1, t2)
    vert_planes = (v0, v1, v2)

    for i in range(3):
        tri = tri_planes[i]

        def body(n, _):
            base = (wid + n * NW) * CH
            pltpu.sync_copy(tri.at[pl.ds(base, CH)], idx_v)
            for k in range(3):
                pltpu.async_copy(vert_planes[k].at[idx_v], row_v, sem_g).wait()
                pltpu.sync_copy(row_v, out_hbm.at[i].at[k].at[pl.ds(base, CH)])
            return 0

        nloc = (NCH - wid + NW - 1) // NW
        lax.fori_loop(0, nloc, body, 0)


@jax.jit
def _gather(t0, t1, t2, v0, v1, v2):
    mesh = plsc.VectorSubcoreMesh(core_axis_name="c", subcore_axis_name="s")
    fn = pl.kernel(
        _gather_body,
        mesh=mesh,
        compiler_params=pltpu.CompilerParams(use_tc_tiling_on_sc=False),
        out_type=jax.ShapeDtypeStruct((3, 3, NUM_TRIANGLES), jnp.float32),
        scratch_types=[
            pltpu.VMEM((CH,), jnp.int32),
            pltpu.VMEM((CH,), jnp.float32),
            pltpu.SemaphoreType.DMA,
            pltpu.SemaphoreType.DMA,
        ],
    )
    return fn(t0, t1, t2, v0, v1, v2)


def kernel(vertices, triangles):
    tri = triangles.astype(jnp.int32)
    t0, t1, t2 = tri[:, 0], tri[:, 1], tri[:, 2]
    v0, v1, v2 = vertices[:, 0], vertices[:, 1], vertices[:, 2]
    out = _gather(t0, t1, t2, v0, v1, v2)
    cols = [jnp.stack([out[i, k] for k in range(3)], axis=-1) for i in range(3)]
    return jnp.stack(cols, axis=-2)


# speedup vs baseline: 2.7586x; 2.7586x over previous
import jax
import jax.numpy as jnp
from jax import lax
from jax.experimental import pallas as pl
from jax.experimental.pallas import tpu as pltpu
from jax.experimental.pallas import tpu_sc as plsc

NUM_VERTICES = 1_000_000
NUM_TRIANGLES = 2_000_000
LB = 128
NB = NUM_TRIANGLES // LB
CHB = 125
CH = CHB * LB
NCH = NUM_TRIANGLES // CH
NW = 32


def _gather_body(t0, t1, t2, v0, v1, v2, out_hbm, idx_v, row_v, blk_v, sem_g):
    wid = lax.axis_index("s") * 2 + lax.axis_index("c")
    tri_planes = (t0, t1, t2)
    vert_planes = (v0, v1, v2)

    def to_blocks(b, _):
        for l in range(LB // 16):
            blk_v[b, pl.ds(l * 16, 16)] = row_v[pl.ds(b * LB + l * 16, 16)]
        return 0

    for i in range(3):
        tri = tri_planes[i]

        def body(n, _):
            c = wid + n * NW
            pltpu.sync_copy(tri.at[pl.ds(c * CH, CH)], idx_v)
            for k in range(3):
                pltpu.async_copy(vert_planes[k].at[idx_v], row_v, sem_g).wait()
                lax.fori_loop(0, CHB, to_blocks, 0)
                pltpu.sync_copy(
                    blk_v,
                    out_hbm.at[i].at[pl.ds(c * CHB, CHB)].at[:, k, :],
                )
            return 0

        nloc = (NCH - wid + NW - 1) // NW
        lax.fori_loop(0, nloc, body, 0)


@jax.jit
def _gather(t0, t1, t2, v0, v1, v2):
    mesh = plsc.VectorSubcoreMesh(core_axis_name="c", subcore_axis_name="s")
    fn = pl.kernel(
        _gather_body,
        mesh=mesh,
        compiler_params=pltpu.CompilerParams(use_tc_tiling_on_sc=False),
        out_type=jax.ShapeDtypeStruct((3, NB, 4, LB), jnp.float32),
        scratch_types=[
            pltpu.VMEM((CH,), jnp.int32),
            pltpu.VMEM((CH,), jnp.float32),
            pltpu.VMEM((CHB, LB), jnp.float32),
            pltpu.SemaphoreType.DMA,
        ],
    )
    return fn(t0, t1, t2, v0, v1, v2)


def kernel(vertices, triangles):
    tri = triangles.astype(jnp.int32)
    t0, t1, t2 = tri[:, 0], tri[:, 1], tri[:, 2]
    v0, v1, v2 = vertices[:, 0], vertices[:, 1], vertices[:, 2]
    out = _gather(t0, t1, t2, v0, v1, v2)
    return (
        out[:, :, :3, :]
        .transpose(1, 3, 0, 2)
        .reshape(NUM_TRIANGLES, 3, 3)
    )


# speedup vs baseline: 2.7949x; 1.0132x over previous
import jax
import jax.numpy as jnp
from jax import lax
from jax.experimental import pallas as pl
from jax.experimental.pallas import tpu as pltpu
from jax.experimental.pallas import tpu_sc as plsc

NUM_VERTICES = 1_000_000
NUM_TRIANGLES = 2_000_000
LB = 128
NB = NUM_TRIANGLES // LB
CHB = 125
CH = CHB * LB
NCH = NUM_TRIANGLES // CH
NW = 32


def _gather_body(
    t0, t1, t2, v0, v1, v2, out_hbm, idx_v, row_a, row_b, blk_v, sem_g
):
    wid = lax.axis_index("s") * 2 + lax.axis_index("c")
    tri_planes = (t0, t1, t2)
    vert_planes = (v0, v1, v2)

    def repack_and_store(row_v, i, k, c):
        def to_blocks(b, _):
            for l in range(LB // 16):
                blk_v[b, pl.ds(l * 16, 16)] = row_v[pl.ds(b * LB + l * 16, 16)]
            return 0

        lax.fori_loop(0, CHB, to_blocks, 0)
        pltpu.sync_copy(
            blk_v,
            out_hbm.at[i].at[pl.ds(c * CHB, CHB)].at[:, k, :],
        )

    for i in range(3):
        tri = tri_planes[i]

        def body(n, _):
            c = wid + n * NW
            pltpu.sync_copy(tri.at[pl.ds(c * CH, CH)], idx_v)
            cp0 = pltpu.async_copy(vert_planes[0].at[idx_v], row_a, sem_g)
            cp0.wait()
            cp1 = pltpu.async_copy(vert_planes[1].at[idx_v], row_b, sem_g)
            repack_and_store(row_a, i, 0, c)
            cp1.wait()
            cp2 = pltpu.async_copy(vert_planes[2].at[idx_v], row_a, sem_g)
            repack_and_store(row_b, i, 1, c)
            cp2.wait()
            repack_and_store(row_a, i, 2, c)
            return 0

        nloc = (NCH - wid + NW - 1) // NW
        lax.fori_loop(0, nloc, body, 0)


@jax.jit
def _gather(t0, t1, t2, v0, v1, v2):
    mesh = plsc.VectorSubcoreMesh(core_axis_name="c", subcore_axis_name="s")
    fn = pl.kernel(
        _gather_body,
        mesh=mesh,
        compiler_params=pltpu.CompilerParams(use_tc_tiling_on_sc=False),
        out_type=jax.ShapeDtypeStruct((3, NB, 4, LB), jnp.float32),
        scratch_types=[
            pltpu.VMEM((CH,), jnp.int32),
            pltpu.VMEM((CH,), jnp.float32),
            pltpu.VMEM((CH,), jnp.float32),
            pltpu.VMEM((CHB, LB), jnp.float32),
            pltpu.SemaphoreType.DMA,
        ],
    )
    return fn(t0, t1, t2, v0, v1, v2)


def kernel(vertices, triangles):
    tri = triangles.astype(jnp.int32)
    t0, t1, t2 = tri[:, 0], tri[:, 1], tri[:, 2]
    v0, v1, v2 = vertices[:, 0], vertices[:, 1], vertices[:, 2]
    out = _gather(t0, t1, t2, v0, v1, v2)
    return (
        out[:, :, :3, :]
        .transpose(1, 3, 0, 2)
        .reshape(NUM_TRIANGLES, 3, 3)
    )
